# baseline (speedup 1.0000x reference)
import functools

import jax
import jax.numpy as jnp
from jax import lax
from jax.experimental import pallas as pl
from jax.experimental.pallas import tpu as pltpu
from jax.experimental.pallas import tpu_sc as plsc
import numpy as np

VOCAB_N = 50000
EMBED_N = 64
SAMP_N = 256
LOG_V1 = float(np.log(VOCAB_N + 1.0))

_GCH = 128


def _threefry2x32_np(k0, k1, x0, x1):
    def rotl(x, d):
        return ((x << np.uint32(d)) | (x >> np.uint32(32 - d))).astype(np.uint32)
    ks0, ks1 = np.uint32(k0), np.uint32(k1)
    ks2 = np.uint32(ks0 ^ ks1 ^ np.uint32(0x1BD11BDA))
    x0 = (x0 + ks0).astype(np.uint32)
    x1 = (x1 + ks1).astype(np.uint32)
    keys = [(ks1, ks2), (ks2, ks0), (ks0, ks1), (ks1, ks2), (ks2, ks0)]
    rots = [[13, 15, 26, 6], [17, 29, 16, 24]]
    for i in range(5):
        for r in rots[i % 2]:
            x0 = (x0 + x1).astype(np.uint32)
            x1 = rotl(x1, r)
            x1 = (x1 ^ x0).astype(np.uint32)
        x0 = (x0 + keys[i][0]).astype(np.uint32)
        x1 = (x1 + keys[i][1] + np.uint32(i + 1)).astype(np.uint32)
    return x0, x1


def _log_uniform_sampled_np():
    iota = np.arange(SAMP_N, dtype=np.uint64)
    x0 = (iota >> np.uint64(32)).astype(np.uint32)
    x1 = (iota & np.uint64(0xFFFFFFFF)).astype(np.uint32)
    r0, r1 = _threefry2x32_np(0, 42, x0, x1)
    bits = (r0 ^ r1).astype(np.uint32)
    u = (np.bitwise_or(np.right_shift(bits, np.uint32(9)),
                       np.uint32(0x3F800000))).view(np.float32) - np.float32(1.0)
    c = np.floor(np.exp(u * np.float32(LOG_V1), dtype=np.float32),
                 dtype=np.float32) - np.float32(1.0)
    return np.clip(c.astype(np.int32), 0, VOCAB_N - 1)


_SAMPLED_NP = _log_uniform_sampled_np()


def _samp_neg_log_ec_np():
    c = _SAMPLED_NP.astype(np.float64)
    p = (np.log(c + 2.0) - np.log(c + 1.0)) / LOG_V1
    ec = -np.expm1(float(SAMP_N) * np.log1p(-p))
    return (-np.log(ec)).astype(np.float32)


def _make_sc_gather(B):
    info = plsc.get_sparse_core_info()
    nw = info.num_cores * info.num_subcores
    bpw = B // nw
    nch = bpw // _GCH
    mesh = plsc.VectorSubcoreMesh(core_axis_name="c", subcore_axis_name="s")

    @functools.partial(
        pl.kernel,
        mesh=mesh,
        compiler_params=pltpu.CompilerParams(use_tc_tiling_on_sc=False),
        out_type=[
            jax.ShapeDtypeStruct((B,), jnp.float32),
        ],
        scratch_types=[
            pltpu.VMEM((bpw,), jnp.int32),
            pltpu.VMEM((bpw, EMBED_N), jnp.float32),
            pltpu.VMEM((bpw,), jnp.float32),
            pltpu.SemaphoreType.DMA,
        ],
    )
    def sc_gather(w_hbm, b_hbm, lab_hbm, tb_out,
                  idx_v, rows_v, bias_v, sem):
        wid = lax.axis_index("s") * info.num_cores + lax.axis_index("c")
        base = wid * bpw
        pltpu.sync_copy(lab_hbm.at[pl.ds(base, bpw)], idx_v)
        handles = []
        for k in range(nch):
            sl = pl.ds(k * _GCH, _GCH)
            handles.append(pltpu.async_copy(
                w_hbm.at[idx_v.at[sl]], rows_v.at[sl], sem))
            handles.append(pltpu.async_copy(
                b_hbm.at[idx_v.at[sl]], bias_v.at[sl], sem))
        for h in handles:
            h.wait()
        pltpu.sync_copy(bias_v, tb_out.at[pl.ds(base, bpw)])

    return sc_gather


def _xent_neg(logits):
    return jnp.maximum(logits, 0.0) + jnp.log(1.0 + jnp.exp(-jnp.abs(logits)))


def _xent_pos(logits):
    return (jnp.maximum(logits, 0.0) - logits
            + jnp.log(1.0 + jnp.exp(-jnp.abs(logits))))


def _log1p_small(x):
    u = 1.0 + x
    return jnp.log(u) + (x - (u - 1.0)) / u


def _neg_log_expected_count(c_f32):
    p = (jnp.log(c_f32 + 2.0) - jnp.log(c_f32 + 1.0)) / LOG_V1
    ec = 1.0 - jnp.exp(float(SAMP_N) * _log1p_small(-p))
    return -jnp.log(ec)


def _tc_samp_body(e_ref, sw_ref, adj_ref, out_ref, *, inv_b):
    i = pl.program_id(0)
    e = e_ref[...]
    sl = lax.dot_general(e, sw_ref[...], (((1,), (1,)), ((), ())),
                         preferred_element_type=jnp.float32)
    sl = sl + adj_ref[...][None, :]
    total = jnp.sum(_xent_neg(sl)) * inv_b

    @pl.when(i == 0)
    def _():
        out_ref[...] = jnp.zeros((1, 1), jnp.float32)

    out_ref[...] += total[None, None]


def _tc_true_body(lab_ref, e_ref, tw_ref, tb_ref, samp_ref, out_ref,
                  *, inv_b):
    i = pl.program_id(0)
    e = e_ref[...]
    tw = tw_ref[...][:, :EMBED_N]
    dot_t = jnp.sum(e * tw, axis=1)
    tl = (dot_t + tb_ref[...]
          + _neg_log_expected_count(lab_ref[...].astype(jnp.float32)))
    total = jnp.sum(_xent_pos(tl)) * inv_b

    @pl.when(i == 0)
    def _():
        out_ref[...] = samp_ref[...]

    out_ref[...] += total[None, None]


def kernel(embedding, nce_weight, nce_bias, target_words):
    B = embedding.shape[0]
    labels = target_words.reshape(-1).astype(jnp.int32)
    samp_nlec = jnp.asarray(_samp_neg_log_ec_np())

    sampled = jnp.asarray(_SAMPLED_NP)
    sw = jnp.take(nce_weight, sampled, axis=0)
    sb = jnp.take(nce_bias, sampled, axis=0)
    adj_s = sb + samp_nlec

    Bb = 1024
    nblocks = B // Bb

    samp_part = pl.pallas_call(
        functools.partial(_tc_samp_body, inv_b=1.0 / B),
        grid=(nblocks,),
        in_specs=[
            pl.BlockSpec((Bb, EMBED_N), lambda i: (i, 0)),
            pl.BlockSpec((SAMP_N, EMBED_N), lambda i: (0, 0)),
            pl.BlockSpec((SAMP_N,), lambda i: (0,)),
        ],
        out_specs=pl.BlockSpec((1, 1), lambda i: (0, 0)),
        out_shape=jax.ShapeDtypeStruct((1, 1), jnp.float32),
        compiler_params=pltpu.CompilerParams(
            dimension_semantics=("arbitrary",)),
    )(embedding, sw, adj_s)

    sc_gather = _make_sc_gather(B)
    tb = jax.tree.leaves(sc_gather(nce_weight, nce_bias, labels))[0]
    tw = jnp.zeros((B, 128), jnp.float32)

    out = pl.pallas_call(
        functools.partial(_tc_true_body, inv_b=1.0 / B),
        grid=(nblocks,),
        in_specs=[
            pl.BlockSpec((Bb,), lambda i: (i,)),
            pl.BlockSpec((Bb, EMBED_N), lambda i: (i, 0)),
            pl.BlockSpec((Bb, 128), lambda i: (i, 0)),
            pl.BlockSpec((Bb,), lambda i: (i,)),
            pl.BlockSpec((1, 1), lambda i: (0, 0)),
        ],
        out_specs=pl.BlockSpec((1, 1), lambda i: (0, 0)),
        out_shape=jax.ShapeDtypeStruct((1, 1), jnp.float32),
        compiler_params=pltpu.CompilerParams(
            dimension_semantics=("arbitrary",)),
    )(labels, embedding, tw, tb, samp_part)
    return out[0, 0]


# speedup vs baseline: 1.2867x; 1.2867x over previous
import functools

import jax
import jax.numpy as jnp
from jax import lax
from jax.experimental import pallas as pl
from jax.experimental.pallas import tpu as pltpu
from jax.experimental.pallas import tpu_sc as plsc
import numpy as np

VOCAB_N = 50000
EMBED_N = 64
SAMP_N = 256
LOG_V1 = float(np.log(VOCAB_N + 1.0))

_GCH = 128


def _threefry2x32_np(k0, k1, x0, x1):
    def rotl(x, d):
        return ((x << np.uint32(d)) | (x >> np.uint32(32 - d))).astype(np.uint32)
    ks0, ks1 = np.uint32(k0), np.uint32(k1)
    ks2 = np.uint32(ks0 ^ ks1 ^ np.uint32(0x1BD11BDA))
    x0 = (x0 + ks0).astype(np.uint32)
    x1 = (x1 + ks1).astype(np.uint32)
    keys = [(ks1, ks2), (ks2, ks0), (ks0, ks1), (ks1, ks2), (ks2, ks0)]
    rots = [[13, 15, 26, 6], [17, 29, 16, 24]]
    for i in range(5):
        for r in rots[i % 2]:
            x0 = (x0 + x1).astype(np.uint32)
            x1 = rotl(x1, r)
            x1 = (x1 ^ x0).astype(np.uint32)
        x0 = (x0 + keys[i][0]).astype(np.uint32)
        x1 = (x1 + keys[i][1] + np.uint32(i + 1)).astype(np.uint32)
    return x0, x1


def _log_uniform_sampled_np():
    iota = np.arange(SAMP_N, dtype=np.uint64)
    x0 = (iota >> np.uint64(32)).astype(np.uint32)
    x1 = (iota & np.uint64(0xFFFFFFFF)).astype(np.uint32)
    r0, r1 = _threefry2x32_np(0, 42, x0, x1)
    bits = (r0 ^ r1).astype(np.uint32)
    u = (np.bitwise_or(np.right_shift(bits, np.uint32(9)),
                       np.uint32(0x3F800000))).view(np.float32) - np.float32(1.0)
    c = np.floor(np.exp(u * np.float32(LOG_V1), dtype=np.float32),
                 dtype=np.float32) - np.float32(1.0)
    return np.clip(c.astype(np.int32), 0, VOCAB_N - 1)


_SAMPLED_NP = _log_uniform_sampled_np()


def _samp_neg_log_ec_np():
    c = _SAMPLED_NP.astype(np.float64)
    p = (np.log(c + 2.0) - np.log(c + 1.0)) / LOG_V1
    ec = -np.expm1(float(SAMP_N) * np.log1p(-p))
    return (-np.log(ec)).astype(np.float32)


def _make_sc_gather(B):
    info = plsc.get_sparse_core_info()
    nw = info.num_cores * info.num_subcores
    bpw = B // nw
    spw = SAMP_N // nw
    nch = bpw // _GCH
    mesh = plsc.VectorSubcoreMesh(core_axis_name="c", subcore_axis_name="s")

    @functools.partial(
        pl.kernel,
        mesh=mesh,
        compiler_params=pltpu.CompilerParams(use_tc_tiling_on_sc=False),
        out_type=[
            jax.ShapeDtypeStruct((B, 128), jnp.float32),
            jax.ShapeDtypeStruct((B,), jnp.float32),
            jax.ShapeDtypeStruct((SAMP_N, 128), jnp.float32),
            jax.ShapeDtypeStruct((SAMP_N,), jnp.float32),
        ],
        scratch_types=[
            pltpu.VMEM((bpw,), jnp.int32),
            pltpu.VMEM((bpw, EMBED_N), jnp.float32),
            pltpu.VMEM((bpw,), jnp.float32),
            pltpu.VMEM((spw,), jnp.int32),
            pltpu.VMEM((spw, EMBED_N), jnp.float32),
            pltpu.VMEM((spw,), jnp.float32),
            pltpu.SemaphoreType.DMA,
        ],
    )
    def sc_gather(w_hbm, b_hbm, lab_hbm, samp_hbm,
                  tw_out, tb_out, sw_out, sb_out,
                  idx_v, rows_v, bias_v, sidx_v, srows_v, sbias_v, sem):
        wid = lax.axis_index("s") * info.num_cores + lax.axis_index("c")
        base = wid * bpw
        pltpu.sync_copy(lab_hbm.at[pl.ds(base, bpw)], idx_v)
        sbase = wid * spw
        pltpu.sync_copy(samp_hbm.at[pl.ds(sbase, spw)], sidx_v)
        handles = []
        for k in range(nch):
            sl = pl.ds(k * _GCH, _GCH)
            handles.append(pltpu.async_copy(
                w_hbm.at[idx_v.at[sl]], rows_v.at[sl], sem))
            handles.append(pltpu.async_copy(
                b_hbm.at[idx_v.at[sl]], bias_v.at[sl], sem))
        handles.append(pltpu.async_copy(w_hbm.at[sidx_v], srows_v, sem))
        handles.append(pltpu.async_copy(b_hbm.at[sidx_v], sbias_v, sem))
        for h in handles:
            h.wait()
        pltpu.sync_copy(rows_v,
                        tw_out.at[pl.ds(base, bpw), pl.ds(0, EMBED_N)])
        pltpu.sync_copy(bias_v, tb_out.at[pl.ds(base, bpw)])
        pltpu.sync_copy(srows_v,
                        sw_out.at[pl.ds(sbase, spw), pl.ds(0, EMBED_N)])
        pltpu.sync_copy(sbias_v, sb_out.at[pl.ds(sbase, spw)])

    return sc_gather


def _xent_neg(logits):
    return jnp.maximum(logits, 0.0) + jnp.log(1.0 + jnp.exp(-jnp.abs(logits)))


def _xent_pos(logits):
    return (jnp.maximum(logits, 0.0) - logits
            + jnp.log(1.0 + jnp.exp(-jnp.abs(logits))))


def _log1p_small(x):
    u = 1.0 + x
    return jnp.log(u) + (x - (u - 1.0)) / u


def _neg_log_expected_count(c_f32):
    p = (jnp.log(c_f32 + 2.0) - jnp.log(c_f32 + 1.0)) / LOG_V1
    ec = 1.0 - jnp.exp(float(SAMP_N) * _log1p_small(-p))
    return -jnp.log(ec)


def _tc_body(lab_ref, e_ref, tw_ref, tb_ref, sw_ref, sb_ref, nlec_ref,
             out_ref, *, inv_b):
    i = pl.program_id(0)
    e = e_ref[...]

    tw = tw_ref[...][:, :EMBED_N]
    dot_t = jnp.sum(e * tw, axis=1)
    tl = (dot_t + tb_ref[...]
          + _neg_log_expected_count(lab_ref[...].astype(jnp.float32)))
    total = jnp.sum(_xent_pos(tl))

    sw = sw_ref[...][:, :EMBED_N]
    adj = sb_ref[...] + nlec_ref[...]
    sl = lax.dot_general(e, sw, (((1,), (1,)), ((), ())),
                         preferred_element_type=jnp.float32)
    sl = sl + adj[None, :]
    total = (total + jnp.sum(_xent_neg(sl))) * inv_b

    @pl.when(i == 0)
    def _():
        out_ref[...] = jnp.zeros((1, 1), jnp.float32)

    out_ref[...] += total[None, None]


def kernel(embedding, nce_weight, nce_bias, target_words):
    B = embedding.shape[0]
    labels = target_words.reshape(-1).astype(jnp.int32)
    sampled = jnp.asarray(_SAMPLED_NP)
    samp_nlec = jnp.asarray(_samp_neg_log_ec_np())

    sc_gather = _make_sc_gather(B)
    tw, tb, sw, sb = sc_gather(nce_weight, nce_bias, labels, sampled)

    Bb = 1024
    nblocks = B // Bb
    out = pl.pallas_call(
        functools.partial(_tc_body, inv_b=1.0 / B),
        grid=(nblocks,),
        in_specs=[
            pl.BlockSpec((Bb,), lambda i: (i,)),
            pl.BlockSpec((Bb, EMBED_N), lambda i: (i, 0)),
            pl.BlockSpec((Bb, 128), lambda i: (i, 0)),
            pl.BlockSpec((Bb,), lambda i: (i,)),
            pl.BlockSpec((SAMP_N, 128), lambda i: (0, 0)),
            pl.BlockSpec((SAMP_N,), lambda i: (0,)),
            pl.BlockSpec((SAMP_N,), lambda i: (0,)),
        ],
        out_specs=pl.BlockSpec((1, 1), lambda i: (0, 0)),
        out_shape=jax.ShapeDtypeStruct((1, 1), jnp.float32),
        compiler_params=pltpu.CompilerParams(
            dimension_semantics=("arbitrary",)),
    )(labels, embedding, tw, tb, sw, sb, samp_nlec)
    return out[0, 0]


# speedup vs baseline: 1.2917x; 1.0038x over previous
import functools

import jax
import jax.numpy as jnp
from jax import lax
from jax.experimental import pallas as pl
from jax.experimental.pallas import tpu as pltpu
from jax.experimental.pallas import tpu_sc as plsc
import numpy as np

VOCAB_N = 50000
EMBED_N = 64
SAMP_N = 256
LOG_V1 = float(np.log(VOCAB_N + 1.0))

_GCH = 128


def _threefry2x32_np(k0, k1, x0, x1):
    def rotl(x, d):
        return ((x << np.uint32(d)) | (x >> np.uint32(32 - d))).astype(np.uint32)
    ks0, ks1 = np.uint32(k0), np.uint32(k1)
    ks2 = np.uint32(ks0 ^ ks1 ^ np.uint32(0x1BD11BDA))
    x0 = (x0 + ks0).astype(np.uint32)
    x1 = (x1 + ks1).astype(np.uint32)
    keys = [(ks1, ks2), (ks2, ks0), (ks0, ks1), (ks1, ks2), (ks2, ks0)]
    rots = [[13, 15, 26, 6], [17, 29, 16, 24]]
    for i in range(5):
        for r in rots[i % 2]:
            x0 = (x0 + x1).astype(np.uint32)
            x1 = rotl(x1, r)
            x1 = (x1 ^ x0).astype(np.uint32)
        x0 = (x0 + keys[i][0]).astype(np.uint32)
        x1 = (x1 + keys[i][1] + np.uint32(i + 1)).astype(np.uint32)
    return x0, x1


def _log_uniform_sampled_np():
    iota = np.arange(SAMP_N, dtype=np.uint64)
    x0 = (iota >> np.uint64(32)).astype(np.uint32)
    x1 = (iota & np.uint64(0xFFFFFFFF)).astype(np.uint32)
    r0, r1 = _threefry2x32_np(0, 42, x0, x1)
    bits = (r0 ^ r1).astype(np.uint32)
    u = (np.bitwise_or(np.right_shift(bits, np.uint32(9)),
                       np.uint32(0x3F800000))).view(np.float32) - np.float32(1.0)
    c = np.floor(np.exp(u * np.float32(LOG_V1), dtype=np.float32),
                 dtype=np.float32) - np.float32(1.0)
    return np.clip(c.astype(np.int32), 0, VOCAB_N - 1)


_SAMPLED_NP = _log_uniform_sampled_np()


def _samp_neg_log_ec_np():
    c = _SAMPLED_NP.astype(np.float64)
    p = (np.log(c + 2.0) - np.log(c + 1.0)) / LOG_V1
    ec = -np.expm1(float(SAMP_N) * np.log1p(-p))
    return (-np.log(ec)).astype(np.float32)


def _make_sc_gather(B):
    info = plsc.get_sparse_core_info()
    nw = info.num_cores * info.num_subcores
    bpw = B // nw
    spw = SAMP_N // nw
    nch = bpw // _GCH
    mesh = plsc.VectorSubcoreMesh(core_axis_name="c", subcore_axis_name="s")

    @functools.partial(
        pl.kernel,
        mesh=mesh,
        compiler_params=pltpu.CompilerParams(use_tc_tiling_on_sc=False),
        out_type=[
            jax.ShapeDtypeStruct((B, 128), jnp.float32),
            jax.ShapeDtypeStruct((B,), jnp.float32),
            jax.ShapeDtypeStruct((SAMP_N, 128), jnp.float32),
            jax.ShapeDtypeStruct((SAMP_N,), jnp.float32),
        ],
        scratch_types=[
            pltpu.VMEM((bpw,), jnp.int32),
            pltpu.VMEM((bpw, EMBED_N), jnp.float32),
            pltpu.VMEM((bpw,), jnp.float32),
            pltpu.VMEM((spw,), jnp.int32),
            pltpu.VMEM((spw, EMBED_N), jnp.float32),
            pltpu.VMEM((spw,), jnp.float32),
            pltpu.SemaphoreType.DMA,
        ],
    )
    def sc_gather(w_hbm, b_hbm, lab_hbm, samp_hbm,
                  tw_out, tb_out, sw_out, sb_out,
                  idx_v, rows_v, bias_v, sidx_v, srows_v, sbias_v, sem):
        wid = lax.axis_index("s") * info.num_cores + lax.axis_index("c")
        base = wid * bpw
        pltpu.sync_copy(lab_hbm.at[pl.ds(base, bpw)], idx_v)
        sbase = wid * spw
        pltpu.sync_copy(samp_hbm.at[pl.ds(sbase, spw)], sidx_v)
        handles = []
        for k in range(nch):
            sl = pl.ds(k * _GCH, _GCH)
            handles.append(pltpu.async_copy(
                w_hbm.at[idx_v.at[sl]], rows_v.at[sl], sem))
            handles.append(pltpu.async_copy(
                b_hbm.at[idx_v.at[sl]], bias_v.at[sl], sem))
        handles.append(pltpu.async_copy(w_hbm.at[sidx_v], srows_v, sem))
        handles.append(pltpu.async_copy(b_hbm.at[sidx_v], sbias_v, sem))
        for h in handles:
            h.wait()
        pltpu.sync_copy(rows_v,
                        tw_out.at[pl.ds(base, bpw), pl.ds(0, EMBED_N)])
        pltpu.sync_copy(bias_v, tb_out.at[pl.ds(base, bpw)])
        pltpu.sync_copy(srows_v,
                        sw_out.at[pl.ds(sbase, spw), pl.ds(0, EMBED_N)])
        pltpu.sync_copy(sbias_v, sb_out.at[pl.ds(sbase, spw)])

    return sc_gather


def _xent_neg(logits):
    return jnp.maximum(logits, 0.0) + jnp.log(1.0 + jnp.exp(-jnp.abs(logits)))


def _xent_pos(logits):
    return (jnp.maximum(logits, 0.0) - logits
            + jnp.log(1.0 + jnp.exp(-jnp.abs(logits))))


def _log1p_small(x):
    u = 1.0 + x
    return jnp.log(u) + (x - (u - 1.0)) / u


def _neg_log_expected_count(c_f32):
    p = (jnp.log(c_f32 + 2.0) - jnp.log(c_f32 + 1.0)) / LOG_V1
    ec = 1.0 - jnp.exp(float(SAMP_N) * _log1p_small(-p))
    return -jnp.log(ec)


def _tc_body(lab_ref, e_ref, tw_ref, tb_ref, sw_ref, sb_ref, nlec_ref,
             out_ref, *, inv_b):
    i = pl.program_id(0)
    e = e_ref[...]

    tw = tw_ref[...][:, :EMBED_N]
    dot_t = jnp.sum(e * tw, axis=1)
    tl = (dot_t + tb_ref[...]
          + _neg_log_expected_count(lab_ref[...].astype(jnp.float32)))
    total = jnp.sum(_xent_pos(tl))

    sw = sw_ref[...][:, :EMBED_N]
    adj = sb_ref[...] + nlec_ref[...]
    sl = lax.dot_general(e, sw, (((1,), (1,)), ((), ())),
                         preferred_element_type=jnp.float32)
    sl = sl + adj[None, :]
    total = (total + jnp.sum(_xent_neg(sl))) * inv_b

    @pl.when(i == 0)
    def _():
        out_ref[...] = jnp.zeros((1, 1), jnp.float32)

    out_ref[...] += total[None, None]


def kernel(embedding, nce_weight, nce_bias, target_words):
    B = embedding.shape[0]
    labels = target_words.reshape(-1).astype(jnp.int32)
    sampled = jnp.asarray(_SAMPLED_NP)
    samp_nlec = jnp.asarray(_samp_neg_log_ec_np())

    w_lin = lax.optimization_barrier(nce_weight.reshape(-1))
    w2 = w_lin.reshape(VOCAB_N, EMBED_N)

    sc_gather = _make_sc_gather(B)
    tw, tb, sw, sb = sc_gather(w2, nce_bias, labels, sampled)

    Bb = 1024
    nblocks = B // Bb
    out = pl.pallas_call(
        functools.partial(_tc_body, inv_b=1.0 / B),
        grid=(nblocks,),
        in_specs=[
            pl.BlockSpec((Bb,), lambda i: (i,)),
            pl.BlockSpec((Bb, EMBED_N), lambda i: (i, 0)),
            pl.BlockSpec((Bb, 128), lambda i: (i, 0)),
            pl.BlockSpec((Bb,), lambda i: (i,)),
            pl.BlockSpec((SAMP_N, 128), lambda i: (0, 0)),
            pl.BlockSpec((SAMP_N,), lambda i: (0,)),
            pl.BlockSpec((SAMP_N,), lambda i: (0,)),
        ],
        out_specs=pl.BlockSpec((1, 1), lambda i: (0, 0)),
        out_shape=jax.ShapeDtypeStruct((1, 1), jnp.float32),
        compiler_params=pltpu.CompilerParams(
            dimension_semantics=("arbitrary",)),
    )(labels, embedding, tw, tb, sw, sb, samp_nlec)
    return out[0, 0]


# speedup vs baseline: 1.3622x; 1.0546x over previous
import functools

import jax
import jax.numpy as jnp
from jax import lax
from jax.experimental import pallas as pl
from jax.experimental.pallas import tpu as pltpu
from jax.experimental.pallas import tpu_sc as plsc
import numpy as np

VOCAB_N = 50000
EMBED_N = 64
SAMP_N = 256
LOG_V1 = float(np.log(VOCAB_N + 1.0))

_GCH = 128


def _threefry2x32_np(k0, k1, x0, x1):
    def rotl(x, d):
        return ((x << np.uint32(d)) | (x >> np.uint32(32 - d))).astype(np.uint32)
    ks0, ks1 = np.uint32(k0), np.uint32(k1)
    ks2 = np.uint32(ks0 ^ ks1 ^ np.uint32(0x1BD11BDA))
    x0 = (x0 + ks0).astype(np.uint32)
    x1 = (x1 + ks1).astype(np.uint32)
    keys = [(ks1, ks2), (ks2, ks0), (ks0, ks1), (ks1, ks2), (ks2, ks0)]
    rots = [[13, 15, 26, 6], [17, 29, 16, 24]]
    for i in range(5):
        for r in rots[i % 2]:
            x0 = (x0 + x1).astype(np.uint32)
            x1 = rotl(x1, r)
            x1 = (x1 ^ x0).astype(np.uint32)
        x0 = (x0 + keys[i][0]).astype(np.uint32)
        x1 = (x1 + keys[i][1] + np.uint32(i + 1)).astype(np.uint32)
    return x0, x1


def _log_uniform_sampled_np():
    iota = np.arange(SAMP_N, dtype=np.uint64)
    x0 = (iota >> np.uint64(32)).astype(np.uint32)
    x1 = (iota & np.uint64(0xFFFFFFFF)).astype(np.uint32)
    r0, r1 = _threefry2x32_np(0, 42, x0, x1)
    bits = (r0 ^ r1).astype(np.uint32)
    u = (np.bitwise_or(np.right_shift(bits, np.uint32(9)),
                       np.uint32(0x3F800000))).view(np.float32) - np.float32(1.0)
    c = np.floor(np.exp(u * np.float32(LOG_V1), dtype=np.float32),
                 dtype=np.float32) - np.float32(1.0)
    return np.clip(c.astype(np.int32), 0, VOCAB_N - 1)


_SAMPLED_NP = _log_uniform_sampled_np()


def _samp_neg_log_ec_np():
    c = _SAMPLED_NP.astype(np.float64)
    p = (np.log(c + 2.0) - np.log(c + 1.0)) / LOG_V1
    ec = -np.expm1(float(SAMP_N) * np.log1p(-p))
    return (-np.log(ec)).astype(np.float32)


def _make_sc_gather(B):
    info = plsc.get_sparse_core_info()
    nw = info.num_cores * info.num_subcores
    bpw = B // nw
    spw = SAMP_N // nw
    nch = bpw // _GCH
    mesh = plsc.VectorSubcoreMesh(core_axis_name="c", subcore_axis_name="s")

    @functools.partial(
        pl.kernel,
        mesh=mesh,
        compiler_params=pltpu.CompilerParams(use_tc_tiling_on_sc=False),
        out_type=[
            jax.ShapeDtypeStruct((B, 128), jnp.float32),
            jax.ShapeDtypeStruct((B,), jnp.float32),
            jax.ShapeDtypeStruct((SAMP_N, 128), jnp.float32),
            jax.ShapeDtypeStruct((SAMP_N,), jnp.float32),
        ],
        scratch_types=[
            pltpu.VMEM((bpw,), jnp.int32),
            pltpu.VMEM((bpw, EMBED_N), jnp.float32),
            pltpu.VMEM((bpw,), jnp.float32),
            pltpu.VMEM((spw,), jnp.int32),
            pltpu.VMEM((spw, EMBED_N), jnp.float32),
            pltpu.VMEM((spw,), jnp.float32),
            pltpu.SemaphoreType.DMA,
        ],
    )
    def sc_gather(w_hbm, b_hbm, lab_hbm, samp_hbm,
                  tw_out, tb_out, sw_out, sb_out,
                  idx_v, rows_v, bias_v, sidx_v, srows_v, sbias_v, sem):
        wid = lax.axis_index("s") * info.num_cores + lax.axis_index("c")
        base = wid * bpw
        pltpu.sync_copy(lab_hbm.at[pl.ds(base, bpw)], idx_v)
        sbase = wid * spw
        pltpu.sync_copy(samp_hbm.at[pl.ds(sbase, spw)], sidx_v)
        handles = []
        for k in range(nch):
            sl = pl.ds(k * _GCH, _GCH)
            handles.append(pltpu.async_copy(
                w_hbm.at[idx_v.at[sl]], rows_v.at[sl], sem))
            handles.append(pltpu.async_copy(
                b_hbm.at[idx_v.at[sl]], bias_v.at[sl], sem))
        handles.append(pltpu.async_copy(w_hbm.at[sidx_v], srows_v, sem))
        handles.append(pltpu.async_copy(b_hbm.at[sidx_v], sbias_v, sem))
        for h in handles:
            h.wait()
        pltpu.sync_copy(rows_v,
                        tw_out.at[pl.ds(base, bpw), pl.ds(0, EMBED_N)])
        pltpu.sync_copy(bias_v, tb_out.at[pl.ds(base, bpw)])
        pltpu.sync_copy(srows_v,
                        sw_out.at[pl.ds(sbase, spw), pl.ds(0, EMBED_N)])
        pltpu.sync_copy(sbias_v, sb_out.at[pl.ds(sbase, spw)])

    return sc_gather


def _xent_neg(logits):
    return jnp.maximum(logits, 0.0) + jnp.log(1.0 + jnp.exp(-jnp.abs(logits)))


def _xent_pos(logits):
    return (jnp.maximum(logits, 0.0) - logits
            + jnp.log(1.0 + jnp.exp(-jnp.abs(logits))))


def _log1p_small(x):
    u = 1.0 + x
    return jnp.log(u) + (x - (u - 1.0)) / u


def _neg_log_expected_count(c_f32):
    p = (jnp.log(c_f32 + 2.0) - jnp.log(c_f32 + 1.0)) / LOG_V1
    ec = 1.0 - jnp.exp(float(SAMP_N) * _log1p_small(-p))
    return -jnp.log(ec)


def _tc_body(lab_ref, e_ref, tw_ref, tb_ref, sw_ref, sb_ref, nlec_ref,
             out_ref, *, inv_b):
    i = pl.program_id(0)
    e = e_ref[...]

    tw = tw_ref[...][:, :EMBED_N]
    dot_t = jnp.sum(e * tw, axis=1)
    tl = (dot_t + tb_ref[...]
          + _neg_log_expected_count(lab_ref[...].astype(jnp.float32)))
    total = jnp.sum(jnp.log(1.0 + jnp.exp(-tl)))

    sw = sw_ref[...][:, :EMBED_N]
    adj = sb_ref[...] + nlec_ref[...]
    sl = lax.dot_general(e, sw, (((1,), (1,)), ((), ())),
                         preferred_element_type=jnp.float32)
    sl = sl + adj[None, :]
    total = (total + jnp.sum(jnp.log(1.0 + jnp.exp(sl)))) * inv_b

    @pl.when(i == 0)
    def _():
        out_ref[...] = jnp.zeros((1, 1), jnp.float32)

    out_ref[...] += total[None, None]


def kernel(embedding, nce_weight, nce_bias, target_words):
    B = embedding.shape[0]
    labels = target_words.reshape(-1).astype(jnp.int32)
    sampled = jnp.asarray(_SAMPLED_NP)
    samp_nlec = jnp.asarray(_samp_neg_log_ec_np())

    w_lin = lax.optimization_barrier(nce_weight.reshape(-1))
    w2 = w_lin.reshape(VOCAB_N, EMBED_N)

    sc_gather = _make_sc_gather(B)
    tw, tb, sw, sb = sc_gather(w2, nce_bias, labels, sampled)

    Bb = 2048
    nblocks = B // Bb
    out = pl.pallas_call(
        functools.partial(_tc_body, inv_b=1.0 / B),
        grid=(nblocks,),
        in_specs=[
            pl.BlockSpec((Bb,), lambda i: (i,)),
            pl.BlockSpec((Bb, EMBED_N), lambda i: (i, 0)),
            pl.BlockSpec((Bb, 128), lambda i: (i, 0)),
            pl.BlockSpec((Bb,), lambda i: (i,)),
            pl.BlockSpec((SAMP_N, 128), lambda i: (0, 0)),
            pl.BlockSpec((SAMP_N,), lambda i: (0,)),
            pl.BlockSpec((SAMP_N,), lambda i: (0,)),
        ],
        out_specs=pl.BlockSpec((1, 1), lambda i: (0, 0)),
        out_shape=jax.ShapeDtypeStruct((1, 1), jnp.float32),
        compiler_params=pltpu.CompilerParams(
            dimension_semantics=("arbitrary",)),
    )(labels, embedding, tw, tb, sw, sb, samp_nlec)
    return out[0, 0]


# speedup vs baseline: 1.3686x; 1.0047x over previous
import functools

import jax
import jax.numpy as jnp
from jax import lax
from jax.experimental import pallas as pl
from jax.experimental.pallas import tpu as pltpu
from jax.experimental.pallas import tpu_sc as plsc
import numpy as np

VOCAB_N = 50000
EMBED_N = 64
SAMP_N = 256
LOG_V1 = float(np.log(VOCAB_N + 1.0))

_GCH = 128


def _threefry2x32_np(k0, k1, x0, x1):
    def rotl(x, d):
        return ((x << np.uint32(d)) | (x >> np.uint32(32 - d))).astype(np.uint32)
    ks0, ks1 = np.uint32(k0), np.uint32(k1)
    ks2 = np.uint32(ks0 ^ ks1 ^ np.uint32(0x1BD11BDA))
    x0 = (x0 + ks0).astype(np.uint32)
    x1 = (x1 + ks1).astype(np.uint32)
    keys = [(ks1, ks2), (ks2, ks0), (ks0, ks1), (ks1, ks2), (ks2, ks0)]
    rots = [[13, 15, 26, 6], [17, 29, 16, 24]]
    for i in range(5):
        for r in rots[i % 2]:
            x0 = (x0 + x1).astype(np.uint32)
            x1 = rotl(x1, r)
            x1 = (x1 ^ x0).astype(np.uint32)
        x0 = (x0 + keys[i][0]).astype(np.uint32)
        x1 = (x1 + keys[i][1] + np.uint32(i + 1)).astype(np.uint32)
    return x0, x1


def _log_uniform_sampled_np():
    iota = np.arange(SAMP_N, dtype=np.uint64)
    x0 = (iota >> np.uint64(32)).astype(np.uint32)
    x1 = (iota & np.uint64(0xFFFFFFFF)).astype(np.uint32)
    r0, r1 = _threefry2x32_np(0, 42, x0, x1)
    bits = (r0 ^ r1).astype(np.uint32)
    u = (np.bitwise_or(np.right_shift(bits, np.uint32(9)),
                       np.uint32(0x3F800000))).view(np.float32) - np.float32(1.0)
    c = np.floor(np.exp(u * np.float32(LOG_V1), dtype=np.float32),
                 dtype=np.float32) - np.float32(1.0)
    return np.clip(c.astype(np.int32), 0, VOCAB_N - 1)


_SAMPLED_NP = _log_uniform_sampled_np()


def _samp_neg_log_ec_np():
    c = _SAMPLED_NP.astype(np.float64)
    p = (np.log(c + 2.0) - np.log(c + 1.0)) / LOG_V1
    ec = -np.expm1(float(SAMP_N) * np.log1p(-p))
    return (-np.log(ec)).astype(np.float32)


def _make_sc_gather(B):
    info = plsc.get_sparse_core_info()
    nw = info.num_cores * info.num_subcores
    bpw = B // nw
    spw = SAMP_N // nw
    nch = bpw // _GCH
    mesh = plsc.VectorSubcoreMesh(core_axis_name="c", subcore_axis_name="s")

    @functools.partial(
        pl.kernel,
        mesh=mesh,
        compiler_params=pltpu.CompilerParams(use_tc_tiling_on_sc=False),
        out_type=[
            jax.ShapeDtypeStruct((B, 128), jnp.float32),
            jax.ShapeDtypeStruct((B,), jnp.float32),
            jax.ShapeDtypeStruct((SAMP_N, 128), jnp.float32),
            jax.ShapeDtypeStruct((SAMP_N,), jnp.float32),
        ],
        scratch_types=[
            pltpu.VMEM((bpw,), jnp.int32),
            pltpu.VMEM((bpw, EMBED_N), jnp.float32),
            pltpu.VMEM((bpw,), jnp.float32),
            pltpu.VMEM((spw,), jnp.int32),
            pltpu.VMEM((spw, EMBED_N), jnp.float32),
            pltpu.VMEM((spw,), jnp.float32),
            pltpu.SemaphoreType.DMA,
        ],
    )
    def sc_gather(w_hbm, b_hbm, lab_hbm, samp_hbm,
                  tw_out, tb_out, sw_out, sb_out,
                  idx_v, rows_v, bias_v, sidx_v, srows_v, sbias_v, sem):
        wid = lax.axis_index("s") * info.num_cores + lax.axis_index("c")
        base = wid * bpw
        pltpu.sync_copy(lab_hbm.at[pl.ds(base, bpw)], idx_v)
        sbase = wid * spw
        pltpu.sync_copy(samp_hbm.at[pl.ds(sbase, spw)], sidx_v)
        handles = []
        for k in range(nch):
            sl = pl.ds(k * _GCH, _GCH)
            handles.append(pltpu.async_copy(
                w_hbm.at[idx_v.at[sl]], rows_v.at[sl], sem))
            handles.append(pltpu.async_copy(
                b_hbm.at[idx_v.at[sl]], bias_v.at[sl], sem))
        handles.append(pltpu.async_copy(w_hbm.at[sidx_v], srows_v, sem))
        handles.append(pltpu.async_copy(b_hbm.at[sidx_v], sbias_v, sem))
        for h in handles:
            h.wait()
        pltpu.sync_copy(rows_v,
                        tw_out.at[pl.ds(base, bpw), pl.ds(0, EMBED_N)])
        pltpu.sync_copy(bias_v, tb_out.at[pl.ds(base, bpw)])
        pltpu.sync_copy(srows_v,
                        sw_out.at[pl.ds(sbase, spw), pl.ds(0, EMBED_N)])
        pltpu.sync_copy(sbias_v, sb_out.at[pl.ds(sbase, spw)])

    return sc_gather


def _xent_neg(logits):
    return jnp.maximum(logits, 0.0) + jnp.log(1.0 + jnp.exp(-jnp.abs(logits)))


def _xent_pos(logits):
    return (jnp.maximum(logits, 0.0) - logits
            + jnp.log(1.0 + jnp.exp(-jnp.abs(logits))))


def _log1p_small(x):
    u = 1.0 + x
    return jnp.log(u) + (x - (u - 1.0)) / u


def _neg_log_expected_count(c_f32):
    p = (jnp.log(c_f32 + 2.0) - jnp.log(c_f32 + 1.0)) / LOG_V1
    ec = 1.0 - jnp.exp(float(SAMP_N) * _log1p_small(-p))
    return -jnp.log(ec)


def _tc_body(lab_ref, e_ref, tw_ref, tb_ref, sw_ref, sb_ref, nlec_ref,
             out_ref, *, inv_b):
    i = pl.program_id(0)
    e = e_ref[...]

    tw = tw_ref[...][:, :EMBED_N]
    dot_t = jnp.sum(e * tw, axis=1)
    tl = (dot_t + tb_ref[...]
          + _neg_log_expected_count(lab_ref[...].astype(jnp.float32)))
    total = jnp.sum(jnp.log(1.0 + jnp.exp(-tl)))

    sw = sw_ref[...][:, :EMBED_N]
    adj = sb_ref[...] + nlec_ref[...]
    sl = lax.dot_general(e, sw, (((1,), (1,)), ((), ())),
                         preferred_element_type=jnp.float32)
    sl = sl + adj[None, :]
    total = (total + jnp.sum(jnp.log(1.0 + jnp.exp(sl)))) * inv_b

    @pl.when(i == 0)
    def _():
        out_ref[...] = jnp.zeros((1, 1), jnp.float32)

    out_ref[...] += total[None, None]


def kernel(embedding, nce_weight, nce_bias, target_words):
    B = embedding.shape[0]
    labels = target_words.reshape(-1).astype(jnp.int32)
    sampled = jnp.asarray(_SAMPLED_NP)
    samp_nlec = jnp.asarray(_samp_neg_log_ec_np())

    w_lin = lax.optimization_barrier(nce_weight.reshape(-1))
    w2 = w_lin.reshape(VOCAB_N, EMBED_N)

    sc_gather = _make_sc_gather(B)
    tw, tb, sw, sb = sc_gather(w2, nce_bias, labels, sampled)

    Bb = 4096
    nblocks = B // Bb
    out = pl.pallas_call(
        functools.partial(_tc_body, inv_b=1.0 / B),
        grid=(nblocks,),
        in_specs=[
            pl.BlockSpec((Bb,), lambda i: (i,)),
            pl.BlockSpec((Bb, EMBED_N), lambda i: (i, 0)),
            pl.BlockSpec((Bb, 128), lambda i: (i, 0)),
            pl.BlockSpec((Bb,), lambda i: (i,)),
            pl.BlockSpec((SAMP_N, 128), lambda i: (0, 0)),
            pl.BlockSpec((SAMP_N,), lambda i: (0,)),
            pl.BlockSpec((SAMP_N,), lambda i: (0,)),
        ],
        out_specs=pl.BlockSpec((1, 1), lambda i: (0, 0)),
        out_shape=jax.ShapeDtypeStruct((1, 1), jnp.float32),
        compiler_params=pltpu.CompilerParams(
            dimension_semantics=("arbitrary",)),
    )(labels, embedding, tw, tb, sw, sb, samp_nlec)
    return out[0, 0]
